# Initial kernel scaffold; baseline (speedup 1.0000x reference)
import jax
import jax.numpy as jnp
from jax.experimental import pallas as pl


def kernel(contact_idx, W_orig_contact, bodypart_idx, W_orig_bodypart, upper_bodypart_idx, W_orig_upper_bodypart, lower_bodypart_idx, W_orig_lower_bodypart, multiple_fouls_idx, W_orig_multiple_fouls, try_to_play_idx, W_orig_try_to_play, touch_ball_idx, W_orig_touch_ball, handball_idx, W_orig_handball, handball_offence_idx, W_orig_handball_offence, offence_standard_idx, W_std_offence, contact_standard_idx, W_std_contact, bodypart_standard_idx, W_std_bodypart, upper_bodypart_standard_idx, W_std_upper_bodypart, lower_bodypart_standard_idx, W_std_lower_bodypart, multiple_fouls_standard_idx, W_std_multiple_fouls, try_to_play_standard_idx, W_std_try_to_play, touch_ball_standard_idx, W_std_touch_ball, handball_standard_idx, W_std_handball, handball_offence_standard_idx, W_std_handball_offence):
    raise NotImplementedError("write your pallas kernel here")


# speedup vs baseline: 1.6063x; 1.6063x over previous
import functools

import jax
import jax.numpy as jnp
from jax import lax
from jax.experimental import pallas as pl
from jax.experimental.pallas import tpu as pltpu
from jax.experimental.pallas import tpu_sc as plsc

B = 16384
NC, NS = 2, 16
NW = NC * NS
BPW = B // NW
CHUNK = 128
NCHUNK = BPW // CHUNK

ORIG_D = [64, 64, 32, 32, 16, 16, 16, 32, 32]
STD_D = [32, 64, 64, 32, 32, 16, 16, 16, 32, 32]
D_ORIG_TOT = sum(ORIG_D)
D_STD_TOT = sum(STD_D)


def _offsets(ds):
    offs, c = [], 0
    for d in ds:
        offs.append(c)
        c += d
    return offs

ORIG_OFF = _offsets(ORIG_D)
STD_OFF = _offsets(STD_D)


def _body(*refs):
    idx_refs = list(refs[0:19])
    tab_refs = list(refs[19:38])
    out_orig, out_std = refs[38], refs[39]
    idx_v = refs[40]
    rows = {64: refs[41], 32: refs[42], 16: refs[43]}
    sem = refs[44]

    wid = lax.axis_index("s") * NC + lax.axis_index("c")
    rbase = wid * NCHUNK
    obase = wid * BPW

    specs = []
    for t in range(9):
        specs.append((idx_refs[t], tab_refs[t], out_orig, ORIG_OFF[t], ORIG_D[t]))
    for t in range(10):
        specs.append((idx_refs[9 + t], tab_refs[9 + t], out_std, STD_OFF[t], STD_D[t]))

    for (idx_hbm, tab_hbm, out_hbm, c0, d) in specs:
        rows_v = rows[d]
        pltpu.sync_copy(idx_hbm.at[pl.ds(rbase, NCHUNK)], idx_v)
        descs = [
            pltpu.async_copy(
                tab_hbm.at[idx_v.at[j]],
                rows_v.at[pl.ds(j * CHUNK, CHUNK)],
                sem,
            )
            for j in range(NCHUNK)
        ]
        for dsc in descs:
            dsc.wait()
        pltpu.sync_copy(rows_v, out_hbm.at[pl.ds(obase, BPW), pl.ds(c0, d)])


@functools.partial(jax.jit, static_argnames=())
def _run(idxs, tabs):
    mesh = plsc.VectorSubcoreMesh(
        core_axis_name="c", subcore_axis_name="s", num_cores=NC, num_subcores=NS
    )
    fn = pl.kernel(
        _body,
        out_type=(
            jax.ShapeDtypeStruct((B, D_ORIG_TOT), jnp.float32),
            jax.ShapeDtypeStruct((B, D_STD_TOT), jnp.float32),
        ),
        mesh=mesh,
        scratch_types=(
            pltpu.VMEM((NCHUNK, CHUNK), jnp.int32),
            pltpu.VMEM((BPW, 64), jnp.float32),
            pltpu.VMEM((BPW, 32), jnp.float32),
            pltpu.VMEM((BPW, 16), jnp.float32),
            pltpu.SemaphoreType.DMA,
        ),
        compiler_params=pltpu.CompilerParams(use_tc_tiling_on_sc=False),
    )
    return fn(*idxs, *tabs)


def kernel(contact_idx, W_orig_contact, bodypart_idx, W_orig_bodypart, upper_bodypart_idx, W_orig_upper_bodypart, lower_bodypart_idx, W_orig_lower_bodypart, multiple_fouls_idx, W_orig_multiple_fouls, try_to_play_idx, W_orig_try_to_play, touch_ball_idx, W_orig_touch_ball, handball_idx, W_orig_handball, handball_offence_idx, W_orig_handball_offence, offence_standard_idx, W_std_offence, contact_standard_idx, W_std_contact, bodypart_standard_idx, W_std_bodypart, upper_bodypart_standard_idx, W_std_upper_bodypart, lower_bodypart_standard_idx, W_std_lower_bodypart, multiple_fouls_standard_idx, W_std_multiple_fouls, try_to_play_standard_idx, W_std_try_to_play, touch_ball_standard_idx, W_std_touch_ball, handball_standard_idx, W_std_handball, handball_offence_standard_idx, W_std_handball_offence):
    idxs = [contact_idx, bodypart_idx, upper_bodypart_idx, lower_bodypart_idx,
            multiple_fouls_idx, try_to_play_idx, touch_ball_idx, handball_idx,
            handball_offence_idx,
            offence_standard_idx, contact_standard_idx, bodypart_standard_idx,
            upper_bodypart_standard_idx, lower_bodypart_standard_idx,
            multiple_fouls_standard_idx, try_to_play_standard_idx,
            touch_ball_standard_idx, handball_standard_idx,
            handball_offence_standard_idx]
    tabs = [W_orig_contact, W_orig_bodypart, W_orig_upper_bodypart,
            W_orig_lower_bodypart, W_orig_multiple_fouls, W_orig_try_to_play,
            W_orig_touch_ball, W_orig_handball, W_orig_handball_offence,
            W_std_offence, W_std_contact, W_std_bodypart, W_std_upper_bodypart,
            W_std_lower_bodypart, W_std_multiple_fouls, W_std_try_to_play,
            W_std_touch_ball, W_std_handball, W_std_handball_offence]
    idxs = [i.reshape(B // CHUNK, CHUNK) for i in idxs]
    return _run(idxs, tabs)
